# baseline (speedup 1.0000x reference)
import functools

import jax
import jax.numpy as jnp
from jax import lax
from jax.experimental import pallas as pl
from jax.experimental.pallas import tpu as pltpu
from jax.experimental.pallas import tpu_sc as plsc

NC = 2
NS = 16
NW = NC * NS
CH = 128

F32 = jnp.float32
I32 = jnp.int32


def _gelu(x):
    return 0.5 * x * (1.0 + lax.erf(x * (2.0 ** -0.5)))


def _cdiv(a, b):
    return (a + b - 1) // b


def _gcd(a, b):
    while b:
        a, b = b, a % b
    return a


def _make_mesh():
    return plsc.VectorSubcoreMesh(
        core_axis_name="c", subcore_axis_name="s", num_cores=NC,
        num_subcores=NS)


@functools.lru_cache(maxsize=None)
def _make_deg_kernel(n_pad, cpw):
    sl = n_pad // NS
    zb = _cdiv(sl, 16) * 16

    @functools.partial(
        pl.kernel,
        out_type=(
            jax.ShapeDtypeStruct((NC * n_pad,), F32),
            jax.ShapeDtypeStruct((NC * n_pad,), F32),
        ),
        mesh=_make_mesh(),
        scratch_types=[
            pltpu.VMEM((cpw, CH), I32),
            pltpu.VMEM((cpw, CH), I32),
            pltpu.VMEM((CH,), F32),
            pltpu.VMEM((zb,), F32),
            pltpu.VMEM_SHARED((n_pad,), F32),
            pltpu.VMEM_SHARED((n_pad,), F32),
            pltpu.SemaphoreType.DMA,
        ],
        compiler_params=pltpu.CompilerParams(use_tc_tiling_on_sc=False),
    )
    def deg_kernel(cols_a, cols_b, out_a, out_b,
                   idx_a, idx_b, ones_v, zeros_v, sh_a, sh_b, sem):
        c = lax.axis_index("c")
        s = lax.axis_index("s")
        wid = s * NC + c
        for i in range(CH // 16):
            ones_v[pl.ds(i * 16, 16)] = jnp.ones((16,), F32)
        for i in range(zb // 16):
            zeros_v[pl.ds(i * 16, 16)] = jnp.zeros((16,), F32)
        pltpu.sync_copy(zeros_v.at[pl.ds(0, sl)], sh_a.at[pl.ds(s * sl, sl)])
        pltpu.sync_copy(zeros_v.at[pl.ds(0, sl)], sh_b.at[pl.ds(s * sl, sl)])
        pltpu.sync_copy(cols_a.at[pl.ds(wid * cpw, cpw)], idx_a)
        pltpu.sync_copy(cols_b.at[pl.ds(wid * cpw, cpw)], idx_b)
        plsc.subcore_barrier()

        lag = 8

        def body(j, carry):
            pltpu.async_copy(ones_v, sh_a.at[idx_a.at[j]], sem, add=True)
            pltpu.async_copy(ones_v, sh_b.at[idx_b.at[j]], sem, add=True)

            @pl.when(j >= lag)
            def _drain():
                pltpu.make_async_copy(
                    ones_v, sh_a.at[idx_a.at[j - lag]], sem).wait()
                pltpu.make_async_copy(
                    ones_v, sh_b.at[idx_b.at[j - lag]], sem).wait()

            return carry

        lax.fori_loop(0, cpw, body, 0)
        for k in range(min(lag, cpw)):
            pltpu.make_async_copy(
                ones_v, sh_a.at[idx_a.at[cpw - lag + k]], sem).wait()
            pltpu.make_async_copy(
                ones_v, sh_b.at[idx_b.at[cpw - lag + k]], sem).wait()
        plsc.subcore_barrier()
        pltpu.sync_copy(sh_a.at[pl.ds(s * sl, sl)], zeros_v.at[pl.ds(0, sl)])
        pltpu.sync_copy(zeros_v.at[pl.ds(0, sl)],
                        out_a.at[pl.ds(c * n_pad + s * sl, sl)])
        pltpu.sync_copy(sh_b.at[pl.ds(s * sl, sl)], zeros_v.at[pl.ds(0, sl)])
        pltpu.sync_copy(zeros_v.at[pl.ds(0, sl)],
                        out_b.at[pl.ds(c * n_pad + s * sl, sl)])

    return deg_kernel


@functools.lru_cache(maxsize=None)
def _make_scat_kernel(n_graphs, n_pad, hid, g0, g1):
    sl = n_pad // NS
    kb = 4
    cpw_max = max(g0, g1) * kb

    @functools.partial(
        pl.kernel,
        out_type=tuple(
            jax.ShapeDtypeStruct((NC * n_pad, hid), F32)
            for _ in range(n_graphs)),
        mesh=_make_mesh(),
        scratch_types=(
            [pltpu.VMEM((cpw_max, CH), I32) for _ in range(2 * n_graphs)]
            + [pltpu.VMEM((2 * kb, CH, hid), F32),
               pltpu.VMEM((sl, hid), F32),
               pltpu.SemaphoreType.DMA,
               pltpu.SemaphoreType.DMA]
            + [pltpu.VMEM_SHARED((n_pad, hid), F32) for _ in range(n_graphs)]
        ),
        compiler_params=pltpu.CompilerParams(use_tc_tiling_on_sc=False),
    )
    def scat_kernel(*refs):
        ins = refs[:3 * n_graphs]
        outs = refs[3 * n_graphs:4 * n_graphs]
        sc = refs[4 * n_graphs:]
        idx_v = sc[:2 * n_graphs]
        bufs = sc[2 * n_graphs]
        zbuf = sc[2 * n_graphs + 1]
        gsem = sc[2 * n_graphs + 2]
        ssem = sc[2 * n_graphs + 3]
        shared = sc[2 * n_graphs + 4:]

        c = lax.axis_index("c")
        s = lax.axis_index("s")

        def zrow(i, carry):
            zbuf[i, :] = jnp.zeros((hid,), F32)
            return carry

        lax.fori_loop(0, sl, zrow, 0)
        for g in range(n_graphs):
            pltpu.sync_copy(zbuf, shared[g].at[pl.ds(s * sl, sl)])
        plsc.subcore_barrier()

        def pipeline(ngrp, chunk_base):
            cpwc = ngrp * kb
            for g_i in range(n_graphs):
                pltpu.sync_copy(ins[3 * g_i].at[pl.ds(chunk_base, cpwc)],
                                idx_v[2 * g_i].at[pl.ds(0, cpwc)])
                pltpu.sync_copy(ins[3 * g_i + 1].at[pl.ds(chunk_base, cpwc)],
                                idx_v[2 * g_i + 1].at[pl.ds(0, cpwc)])
            for g_i in range(n_graphs):
                rows_v = idx_v[2 * g_i]
                cols_v = idx_v[2 * g_i + 1]
                y_hbm = ins[3 * g_i + 2]
                sh = shared[g_i]

                for b in range(kb):
                    pltpu.async_copy(y_hbm.at[cols_v.at[b]], bufs.at[b],
                                     gsem)

                def body(g, carry, rows_v=rows_v, cols_v=cols_v,
                         y_hbm=y_hbm, sh=sh):
                    p = lax.rem(g, 2)
                    off = p * kb
                    noff = (1 - p) * kb
                    base = g * kb
                    for b in range(kb):
                        pltpu.make_async_copy(
                            y_hbm.at[cols_v.at[base + b]],
                            bufs.at[off + b], gsem).wait()

                    @pl.when(g >= 1)
                    def _drain_prev():
                        for b in range(kb):
                            pltpu.make_async_copy(
                                bufs.at[noff + b],
                                sh.at[rows_v.at[base - kb + b]], ssem).wait()

                    @pl.when(g + 1 < ngrp)
                    def _prefetch():
                        for b in range(kb):
                            pltpu.async_copy(
                                y_hbm.at[cols_v.at[base + kb + b]],
                                bufs.at[noff + b], gsem)

                    for b in range(kb):
                        pltpu.async_copy(
                            bufs.at[off + b], sh.at[rows_v.at[base + b]],
                            ssem, add=True)
                    return carry

                lax.fori_loop(0, ngrp, body, 0)
                loff = ((ngrp - 1) % 2) * kb
                for b in range(kb):
                    pltpu.make_async_copy(
                        bufs.at[loff + b],
                        sh.at[rows_v.at[(ngrp - 1) * kb + b]], ssem).wait()

        if g0 > 0:
            @pl.when(c == 0)
            def _core0():
                pipeline(g0, s * (g0 * kb))

        if g1 > 0:
            @pl.when(c == 1)
            def _core1():
                pipeline(g1, NS * g0 * kb + s * (g1 * kb))

        plsc.subcore_barrier()
        for g in range(n_graphs):
            pltpu.sync_copy(shared[g].at[pl.ds(s * sl, sl)], zbuf)
            pltpu.sync_copy(zbuf, outs[g].at[pl.ds(c * n_pad + s * sl, sl)])

    return scat_kernel


def _dot(a, b):
    return jnp.dot(a, b, preferred_element_type=F32)


def _full_spec(shape):
    nd = len(shape)
    return pl.BlockSpec(shape, lambda i, nd=nd: (0,) * nd)


def _row_spec(bn, shape):
    nd = len(shape)
    return pl.BlockSpec((bn,) + shape[1:],
                        lambda i, nd=nd: (i,) + (0,) * (nd - 1))


def _part_spec(bn, hid, npb):
    return (pl.BlockSpec((bn, hid), lambda i: (i, 0)),
            pl.BlockSpec((bn, hid), lambda i, npb=npb: (i + npb, 0)))


def _tc1_body(pad_cnt, cx, wcf, bcf, wcc, bcc, tx, wtf, btf, wt1, bt1,
              dpc0, dpc1, dpt0, dpt1, o_xc, o_yc, o_xt, o_y1,
              o_disc, o_dist):
    row0 = (lax.broadcasted_iota(I32, (dpc0.shape[0], 1), 0) == 0)
    corr = jnp.where(row0 & (pl.program_id(0) == 0),
                     jnp.float32(pad_cnt), 0.0)
    disc = lax.rsqrt(dpc0[...] + dpc1[...] + 1.0 - corr)
    dist = lax.rsqrt(dpt0[...] + dpt1[...] + 1.0 - corr)
    xc = _gelu(_dot(cx[...], wcf[...]) + bcf[...])
    o_xc[...] = xc
    o_yc[...] = (_dot(xc, wcc[...]) + bcc[...]) * disc
    xt = _gelu(_dot(tx[...], wtf[...]) + btf[...])
    o_xt[...] = xt
    o_y1[...] = (_dot(xt, wt1[...]) + bt1[...]) * dist
    o_disc[...] = disc
    o_dist[...] = dist


def _tc2_body(xc, yc, scp0, scp1, disc, wcp, bcp, xt, y1, s1p0, s1p1,
              dist, wt2, bt2, o_xcf, o_x1, o_xt2, o_y2):
    dc = disc[...]
    sc = scp0[...] + scp1[...]
    h = _gelu(dc * (sc + yc[...]))
    xc2 = (xc[...] + h) * 0.5
    o_xcf[...] = _gelu(_dot(xc2, wcp[...]) + bcp[...])
    dt = dist[...]
    s1 = s1p0[...] + s1p1[...]
    x1 = _gelu(dt * (s1 + y1[...]))
    o_x1[...] = x1
    xt2 = xt[...] + x1
    o_xt2[...] = xt2
    o_y2[...] = (_dot(xt2, wt2[...]) + bt2[...]) * dt


def _tc3_body(seq_len, half,
              s2p0, s2p1, y2, dist, xt2, x1, wtp, btp, xcf,
              wih_f_t, whh_f_t, b_f, wih_r_t, whh_r_t, b_r,
              w_attn, b_attn, w_fcc, b_fcc, w_fct, b_fct,
              o_outc, o_outt, o_xtf, hs_scr, z_scr):
    blk = pl.program_id(0)
    dt = dist[...]
    s2 = s2p0[...] + s2p1[...]
    x2 = _gelu(dt * (s2 + y2[...]))
    xt3 = xt2[...] + x2
    xavg = (xt3 + x1[...] + x2) * (1.0 / 3.0)
    xtf = _gelu(_dot(xavg, wtp[...]) + btp[...])
    o_xtf[...] = xtf

    @pl.when(blk == 0)
    def _lstm():
        wihf = wih_f_t[...]
        whhf = whh_f_t[...]
        bf = b_f[...]
        wihr = wih_r_t[...]
        whhr = whh_r_t[...]
        br = b_r[...]

        def cell(xrow, h, c, wih, whh, b):
            g = _dot(xrow, wih) + _dot(h, whh) + b
            i = jax.nn.sigmoid(g[:, 0:half])
            f = jax.nn.sigmoid(g[:, half:2 * half])
            gg = jnp.tanh(g[:, 2 * half:3 * half])
            o = jax.nn.sigmoid(g[:, 3 * half:4 * half])
            c = f * c + i * gg
            h = o * jnp.tanh(c)
            return h, c

        def step(t, carry):
            hf, cf, hr, cr = carry
            xf = o_xtf[pl.ds(t, 1), :]
            xr = o_xtf[pl.ds(seq_len - 1 - t, 1), :]
            hf, cf = cell(xf, hf, cf, wihf, whhf, bf)
            hr, cr = cell(xr, hr, cr, wihr, whhr, br)
            hs_scr[pl.ds(t, 1), 0:half] = hf
            hs_scr[pl.ds(seq_len - 1 - t, 1), half:2 * half] = hr
            return hf, cf, hr, cr

        z0 = jnp.zeros((1, half), F32)
        lax.fori_loop(0, seq_len, step, (z0, z0, z0, z0))

        hseq = hs_scr[0:seq_len, :]
        scores = _dot(hseq, w_attn[...]) + b_attn[...]
        m = jnp.max(scores, axis=0, keepdims=True)
        e = jnp.exp(scores - m)
        a = e / jnp.sum(e, axis=0, keepdims=True)
        z_scr[...] = jnp.sum(a * hseq, axis=0, keepdims=True)

    z = z_scr[...]
    o_outc[...] = _dot(xcf[...] + z, w_fcc[...]) + b_fcc[...]
    o_outt[...] = _dot(xtf, w_fct[...]) + b_fct[...]


def kernel(company_x, transaction_x, company_edge_index_h,
           transaction_edge_index_h,
           W_cffn, b_cffn, W_cconv, b_cconv, W_cpost, b_cpost,
           W_tffn, b_tffn, W_tc1, b_tc1, W_tc2, b_tc2, W_tpost, b_tpost,
           Wih_f, Whh_f, bih_f, bhh_f, Wih_r, Whh_r, bih_r, bhh_r,
           W_attn, b_attn, W_fcc, b_fcc, W_fct, b_fct):
    n = company_x.shape[0]
    d_in = company_x.shape[1]
    hid = W_cffn.shape[1]
    e_c = company_edge_index_h.shape[1]
    e_t = transaction_edge_index_h.shape[1]
    assert e_c == e_t and transaction_x.shape[0] == n
    seq_len = 50
    half = Whh_f.shape[1]

    kb = 4
    gtot = _cdiv(_cdiv(e_c, CH), NS * kb)
    g0 = (gtot * 28) // 40
    g1 = gtot - g0
    nch = NS * gtot * kb
    cpw = nch // NW
    e_pad = nch * CH

    bn = 2000
    grid = n // bn
    assert n % bn == 0
    n_pad = _cdiv(n + 8, NS * 8 * bn // _gcd(NS * 8, bn)) \
        * (NS * 8 * bn // _gcd(NS * 8, bn))
    npb = n_pad // bn

    pad = e_pad - e_c
    scrap = n + (jnp.arange(pad, dtype=I32) % (n_pad - n))

    def prep(ei):
        rows = jnp.concatenate([ei[0], scrap]).reshape(nch, CH)
        colsg = jnp.concatenate(
            [ei[1], jnp.zeros((pad,), I32)]).reshape(nch, CH)
        return rows, colsg

    rows_c, colsg_c = prep(company_edge_index_h)
    rows_t, colsg_t = prep(transaction_edge_index_h)

    degc_p, degt_p = _make_deg_kernel(n_pad, cpw)(colsg_c, colsg_t)
    degc_p = degc_p.reshape(NC * n_pad, 1)
    degt_p = degt_p.reshape(NC * n_pad, 1)

    nh = (n, hid)
    d0, d1 = _part_spec(bn, 1, npb)
    tc1 = pl.pallas_call(
        functools.partial(_tc1_body, pad),
        grid=(grid,),
        in_specs=[
            _row_spec(bn, (n, d_in)), _full_spec(W_cffn.shape),
            _full_spec(b_cffn.shape), _full_spec(W_cconv.shape),
            _full_spec(b_cconv.shape),
            _row_spec(bn, (n, d_in)), _full_spec(W_tffn.shape),
            _full_spec(b_tffn.shape), _full_spec(W_tc1.shape),
            _full_spec(b_tc1.shape),
            d0, d1, d0, d1,
        ],
        out_specs=[_row_spec(bn, nh), _row_spec(bn, nh),
                   _row_spec(bn, nh), _row_spec(bn, nh),
                   _row_spec(bn, (n, 1)), _row_spec(bn, (n, 1))],
        out_shape=[
            jax.ShapeDtypeStruct(nh, F32),
            jax.ShapeDtypeStruct(nh, F32),
            jax.ShapeDtypeStruct(nh, F32),
            jax.ShapeDtypeStruct(nh, F32),
            jax.ShapeDtypeStruct((n, 1), F32),
            jax.ShapeDtypeStruct((n, 1), F32),
        ],
    )
    x_c, y_c, x_t, y_1, dis_c, dis_t = tc1(
        company_x, W_cffn, b_cffn, W_cconv, b_cconv,
        transaction_x, W_tffn, b_tffn, W_tc1, b_tc1,
        degc_p, degc_p, degt_p, degt_p)

    sc_p, s1_p = _make_scat_kernel(2, n_pad, hid, g0, g1)(
        rows_c, colsg_c, y_c, rows_t, colsg_t, y_1)

    p0, p1 = _part_spec(bn, hid, npb)
    tc2 = pl.pallas_call(
        _tc2_body,
        grid=(grid,),
        in_specs=[
            _row_spec(bn, nh), _row_spec(bn, nh), p0, p1,
            _row_spec(bn, (n, 1)), _full_spec(W_cpost.shape),
            _full_spec(b_cpost.shape),
            _row_spec(bn, nh), _row_spec(bn, nh), p0, p1,
            _row_spec(bn, (n, 1)), _full_spec(W_tc2.shape),
            _full_spec(b_tc2.shape),
        ],
        out_specs=[_row_spec(bn, nh)] * 4,
        out_shape=[jax.ShapeDtypeStruct(nh, F32)] * 4,
    )
    x_cf, x_1, x_t2, y_2 = tc2(
        x_c, y_c, sc_p, sc_p, dis_c, W_cpost, b_cpost,
        x_t, y_1, s1_p, s1_p, dis_t, W_tc2, b_tc2)

    (s2_p,) = _make_scat_kernel(1, n_pad, hid, g0, g1)(
        rows_t, colsg_t, y_2)

    wih_f_t = jnp.transpose(Wih_f)
    whh_f_t = jnp.transpose(Whh_f)
    wih_r_t = jnp.transpose(Wih_r)
    whh_r_t = jnp.transpose(Whh_r)
    b_f = bih_f + bhh_f
    b_r = bih_r + bhh_r
    tc3 = pl.pallas_call(
        functools.partial(_tc3_body, seq_len, half),
        grid=(grid,),
        in_specs=[
            p0, p1, _row_spec(bn, nh),
            _row_spec(bn, (n, 1)), _row_spec(bn, nh), _row_spec(bn, nh),
            _full_spec(W_tpost.shape), _full_spec(b_tpost.shape),
            _row_spec(bn, nh),
            _full_spec(wih_f_t.shape), _full_spec(whh_f_t.shape),
            _full_spec(b_f.shape),
            _full_spec(wih_r_t.shape), _full_spec(whh_r_t.shape),
            _full_spec(b_r.shape),
            _full_spec(W_attn.shape), _full_spec(b_attn.shape),
            _full_spec(W_fcc.shape), _full_spec(b_fcc.shape),
            _full_spec(W_fct.shape), _full_spec(b_fct.shape),
        ],
        out_specs=[_row_spec(bn, (n, 2)), _row_spec(bn, (n, 2)),
                   _row_spec(bn, nh)],
        out_shape=[
            jax.ShapeDtypeStruct((n, 2), F32),
            jax.ShapeDtypeStruct((n, 2), F32),
            jax.ShapeDtypeStruct(nh, F32),
        ],
        scratch_shapes=[pltpu.VMEM((seq_len + 6, hid), F32),
                        pltpu.VMEM((1, hid), F32)],
    )
    out_c, out_t, _ = tc3(
        s2_p, s2_p, y_2, dis_t, x_t2, x_1, W_tpost, b_tpost, x_cf,
        wih_f_t, whh_f_t, b_f, wih_r_t, whh_r_t, b_r,
        W_attn, b_attn, W_fcc, b_fcc, W_fct, b_fct)
    return (out_c, out_t)


# speedup vs baseline: 1.0527x; 1.0527x over previous
import functools

import jax
import jax.numpy as jnp
from jax import lax
from jax.experimental import pallas as pl
from jax.experimental.pallas import tpu as pltpu
from jax.experimental.pallas import tpu_sc as plsc

NC = 2
NS = 16
NW = NC * NS
CH = 128

F32 = jnp.float32
I32 = jnp.int32


def _gelu(x):
    return 0.5 * x * (1.0 + lax.erf(x * (2.0 ** -0.5)))


def _cdiv(a, b):
    return (a + b - 1) // b


def _gcd(a, b):
    while b:
        a, b = b, a % b
    return a


def _make_mesh():
    return plsc.VectorSubcoreMesh(
        core_axis_name="c", subcore_axis_name="s", num_cores=NC,
        num_subcores=NS)


@functools.lru_cache(maxsize=None)
def _make_deg_kernel(n_pad, cpw, nch_real):
    sl = n_pad // NS
    zb = _cdiv(sl, 16) * 16

    @functools.partial(
        pl.kernel,
        out_type=(
            jax.ShapeDtypeStruct((NC * n_pad,), F32),
            jax.ShapeDtypeStruct((NC * n_pad,), F32),
        ),
        mesh=_make_mesh(),
        scratch_types=[
            pltpu.VMEM((cpw, CH), I32),
            pltpu.VMEM((cpw, CH), I32),
            pltpu.VMEM((CH,), F32),
            pltpu.VMEM((zb,), F32),
            pltpu.VMEM_SHARED((n_pad,), F32),
            pltpu.VMEM_SHARED((n_pad,), F32),
            pltpu.SemaphoreType.DMA,
        ],
        compiler_params=pltpu.CompilerParams(use_tc_tiling_on_sc=False),
    )
    def deg_kernel(cols_a, cols_b, out_a, out_b,
                   idx_a, idx_b, ones_v, zeros_v, sh_a, sh_b, sem):
        c = lax.axis_index("c")
        s = lax.axis_index("s")
        wid = s * NC + c
        for i in range(CH // 16):
            ones_v[pl.ds(i * 16, 16)] = jnp.ones((16,), F32)
        for i in range(zb // 16):
            zeros_v[pl.ds(i * 16, 16)] = jnp.zeros((16,), F32)
        pltpu.sync_copy(zeros_v.at[pl.ds(0, sl)], sh_a.at[pl.ds(s * sl, sl)])
        pltpu.sync_copy(zeros_v.at[pl.ds(0, sl)], sh_b.at[pl.ds(s * sl, sl)])
        pltpu.sync_copy(cols_a.at[pl.ds(wid * cpw, cpw)], idx_a)
        pltpu.sync_copy(cols_b.at[pl.ds(wid * cpw, cpw)], idx_b)
        plsc.subcore_barrier()

        lag = 8
        rc = jnp.clip(nch_real - wid * cpw, 0, cpw)

        def body(j, carry):
            pltpu.async_copy(ones_v, sh_a.at[idx_a.at[j]], sem, add=True)
            pltpu.async_copy(ones_v, sh_b.at[idx_b.at[j]], sem, add=True)

            @pl.when(j >= lag)
            def _drain():
                pltpu.make_async_copy(
                    ones_v, sh_a.at[idx_a.at[j - lag]], sem).wait()
                pltpu.make_async_copy(
                    ones_v, sh_b.at[idx_b.at[j - lag]], sem).wait()

            return carry

        lax.fori_loop(0, rc, body, 0)

        def tail(k, carry):
            pltpu.make_async_copy(
                ones_v, sh_a.at[idx_a.at[k]], sem).wait()
            pltpu.make_async_copy(
                ones_v, sh_b.at[idx_b.at[k]], sem).wait()
            return carry

        lax.fori_loop(jnp.maximum(rc - lag, 0), rc, tail, 0)
        plsc.subcore_barrier()
        pltpu.sync_copy(sh_a.at[pl.ds(s * sl, sl)], zeros_v.at[pl.ds(0, sl)])
        pltpu.sync_copy(zeros_v.at[pl.ds(0, sl)],
                        out_a.at[pl.ds(c * n_pad + s * sl, sl)])
        pltpu.sync_copy(sh_b.at[pl.ds(s * sl, sl)], zeros_v.at[pl.ds(0, sl)])
        pltpu.sync_copy(zeros_v.at[pl.ds(0, sl)],
                        out_b.at[pl.ds(c * n_pad + s * sl, sl)])

    return deg_kernel


@functools.lru_cache(maxsize=None)
def _make_scat_kernel(n_graphs, n_pad, hid, g0, g1):
    sl = n_pad // NS
    kb = 4
    cpw_max = max(g0, g1) * kb

    @functools.partial(
        pl.kernel,
        out_type=tuple(
            jax.ShapeDtypeStruct((NC * n_pad, hid), F32)
            for _ in range(n_graphs)),
        mesh=_make_mesh(),
        scratch_types=(
            [pltpu.VMEM((cpw_max, CH), I32) for _ in range(2 * n_graphs)]
            + [pltpu.VMEM((2 * kb, CH, hid), F32),
               pltpu.VMEM((sl, hid), F32),
               pltpu.SemaphoreType.DMA,
               pltpu.SemaphoreType.DMA]
            + [pltpu.VMEM_SHARED((n_pad, hid), F32) for _ in range(n_graphs)]
        ),
        compiler_params=pltpu.CompilerParams(use_tc_tiling_on_sc=False),
    )
    def scat_kernel(*refs):
        ins = refs[:3 * n_graphs]
        outs = refs[3 * n_graphs:4 * n_graphs]
        sc = refs[4 * n_graphs:]
        idx_v = sc[:2 * n_graphs]
        bufs = sc[2 * n_graphs]
        zbuf = sc[2 * n_graphs + 1]
        gsem = sc[2 * n_graphs + 2]
        ssem = sc[2 * n_graphs + 3]
        shared = sc[2 * n_graphs + 4:]

        c = lax.axis_index("c")
        s = lax.axis_index("s")

        def zrow(i, carry):
            zbuf[i, :] = jnp.zeros((hid,), F32)
            return carry

        lax.fori_loop(0, sl, zrow, 0)
        for g in range(n_graphs):
            pltpu.sync_copy(zbuf, shared[g].at[pl.ds(s * sl, sl)])
        plsc.subcore_barrier()

        def pipeline(ngrp, chunk_base):
            cpwc = ngrp * kb
            for g_i in range(n_graphs):
                pltpu.sync_copy(ins[3 * g_i].at[pl.ds(chunk_base, cpwc)],
                                idx_v[2 * g_i].at[pl.ds(0, cpwc)])
                pltpu.sync_copy(ins[3 * g_i + 1].at[pl.ds(chunk_base, cpwc)],
                                idx_v[2 * g_i + 1].at[pl.ds(0, cpwc)])
            for g_i in range(n_graphs):
                rows_v = idx_v[2 * g_i]
                cols_v = idx_v[2 * g_i + 1]
                y_hbm = ins[3 * g_i + 2]
                sh = shared[g_i]

                for b in range(kb):
                    pltpu.async_copy(y_hbm.at[cols_v.at[b]], bufs.at[b],
                                     gsem)

                def body(g, carry, rows_v=rows_v, cols_v=cols_v,
                         y_hbm=y_hbm, sh=sh):
                    p = lax.rem(g, 2)
                    off = p * kb
                    noff = (1 - p) * kb
                    base = g * kb
                    for b in range(kb):
                        pltpu.make_async_copy(
                            y_hbm.at[cols_v.at[base + b]],
                            bufs.at[off + b], gsem).wait()

                    @pl.when(g >= 1)
                    def _drain_prev():
                        for b in range(kb):
                            pltpu.make_async_copy(
                                bufs.at[noff + b],
                                sh.at[rows_v.at[base - kb + b]], ssem).wait()

                    @pl.when(g + 1 < ngrp)
                    def _prefetch():
                        for b in range(kb):
                            pltpu.async_copy(
                                y_hbm.at[cols_v.at[base + kb + b]],
                                bufs.at[noff + b], gsem)

                    for b in range(kb):
                        pltpu.async_copy(
                            bufs.at[off + b], sh.at[rows_v.at[base + b]],
                            ssem, add=True)
                    return carry

                lax.fori_loop(0, ngrp, body, 0)
                loff = ((ngrp - 1) % 2) * kb
                for b in range(kb):
                    pltpu.make_async_copy(
                        bufs.at[loff + b],
                        sh.at[rows_v.at[(ngrp - 1) * kb + b]], ssem).wait()

        if g0 > 0:
            @pl.when(c == 0)
            def _core0():
                pipeline(g0, s * (g0 * kb))

        if g1 > 0:
            @pl.when(c == 1)
            def _core1():
                pipeline(g1, NS * g0 * kb + s * (g1 * kb))

        plsc.subcore_barrier()
        for g in range(n_graphs):
            pltpu.sync_copy(shared[g].at[pl.ds(s * sl, sl)], zbuf)
            pltpu.sync_copy(zbuf, outs[g].at[pl.ds(c * n_pad + s * sl, sl)])

    return scat_kernel


def _dot(a, b):
    return jnp.dot(a, b, preferred_element_type=F32)


def _full_spec(shape):
    nd = len(shape)
    return pl.BlockSpec(shape, lambda i, nd=nd: (0,) * nd)


def _row_spec(bn, shape):
    nd = len(shape)
    return pl.BlockSpec((bn,) + shape[1:],
                        lambda i, nd=nd: (i,) + (0,) * (nd - 1))


def _part_spec(bn, hid):
    return pl.BlockSpec((2, bn, hid), lambda i: (0, i, 0))


def _tc1_body(cx, wcf, bcf, wcc, bcc, tx, wtf, btf, wt1, bt1,
              dpc, dpt, o_xc, o_yc, o_xt, o_y1, o_disc, o_dist):
    disc = lax.rsqrt(dpc[:, 0:1] + dpc[:, 1:2] + 1.0)
    dist = lax.rsqrt(dpt[:, 0:1] + dpt[:, 1:2] + 1.0)
    xc = _gelu(_dot(cx[...], wcf[...]) + bcf[...])
    o_xc[...] = xc
    o_yc[...] = (_dot(xc, wcc[...]) + bcc[...]) * disc
    xt = _gelu(_dot(tx[...], wtf[...]) + btf[...])
    o_xt[...] = xt
    o_y1[...] = (_dot(xt, wt1[...]) + bt1[...]) * dist
    o_disc[...] = disc
    o_dist[...] = dist


def _tc2_body(xc, yc, scp, disc, wcp, bcp, xt, y1, s1p,
              dist, wt2, bt2, o_xcf, o_x1, o_xt2, o_y2):
    dc = disc[...]
    sc = scp[0] + scp[1]
    h = _gelu(dc * (sc + yc[...]))
    xc2 = (xc[...] + h) * 0.5
    o_xcf[...] = _gelu(_dot(xc2, wcp[...]) + bcp[...])
    dt = dist[...]
    s1 = s1p[0] + s1p[1]
    x1 = _gelu(dt * (s1 + y1[...]))
    o_x1[...] = x1
    xt2 = xt[...] + x1
    o_xt2[...] = xt2
    o_y2[...] = (_dot(xt2, wt2[...]) + bt2[...]) * dt


def _tc3_body(seq_len, half,
              s2p, y2, dist, xt2, x1, wtp, btp, xcf,
              wih_f_t, whh_f_t, b_f, wih_r_t, whh_r_t, b_r,
              w_attn, b_attn, w_fcc, b_fcc, w_fct, b_fct,
              o_outc, o_outt, o_xtf, hs_scr, z_scr):
    blk = pl.program_id(0)
    dt = dist[...]
    s2 = s2p[0] + s2p[1]
    x2 = _gelu(dt * (s2 + y2[...]))
    xt3 = xt2[...] + x2
    xavg = (xt3 + x1[...] + x2) * (1.0 / 3.0)
    xtf = _gelu(_dot(xavg, wtp[...]) + btp[...])
    o_xtf[...] = xtf

    @pl.when(blk == 0)
    def _lstm():
        wihf = wih_f_t[...]
        whhf = whh_f_t[...]
        bf = b_f[...]
        wihr = wih_r_t[...]
        whhr = whh_r_t[...]
        br = b_r[...]

        def cell(xrow, h, c, wih, whh, b):
            g = _dot(xrow, wih) + _dot(h, whh) + b
            i = jax.nn.sigmoid(g[:, 0:half])
            f = jax.nn.sigmoid(g[:, half:2 * half])
            gg = jnp.tanh(g[:, 2 * half:3 * half])
            o = jax.nn.sigmoid(g[:, 3 * half:4 * half])
            c = f * c + i * gg
            h = o * jnp.tanh(c)
            return h, c

        def step(t, carry):
            hf, cf, hr, cr = carry
            xf = o_xtf[pl.ds(t, 1), :]
            xr = o_xtf[pl.ds(seq_len - 1 - t, 1), :]
            hf, cf = cell(xf, hf, cf, wihf, whhf, bf)
            hr, cr = cell(xr, hr, cr, wihr, whhr, br)
            hs_scr[pl.ds(t, 1), 0:half] = hf
            hs_scr[pl.ds(seq_len - 1 - t, 1), half:2 * half] = hr
            return hf, cf, hr, cr

        z0 = jnp.zeros((1, half), F32)
        lax.fori_loop(0, seq_len, step, (z0, z0, z0, z0))

        hseq = hs_scr[0:seq_len, :]
        scores = _dot(hseq, w_attn[...]) + b_attn[...]
        m = jnp.max(scores, axis=0, keepdims=True)
        e = jnp.exp(scores - m)
        a = e / jnp.sum(e, axis=0, keepdims=True)
        z_scr[...] = jnp.sum(a * hseq, axis=0, keepdims=True)

    z = z_scr[...]
    o_outc[...] = _dot(xcf[...] + z, w_fcc[...]) + b_fcc[...]
    o_outt[...] = _dot(xtf, w_fct[...]) + b_fct[...]


def kernel(company_x, transaction_x, company_edge_index_h,
           transaction_edge_index_h,
           W_cffn, b_cffn, W_cconv, b_cconv, W_cpost, b_cpost,
           W_tffn, b_tffn, W_tc1, b_tc1, W_tc2, b_tc2, W_tpost, b_tpost,
           Wih_f, Whh_f, bih_f, bhh_f, Wih_r, Whh_r, bih_r, bhh_r,
           W_attn, b_attn, W_fcc, b_fcc, W_fct, b_fct):
    n = company_x.shape[0]
    d_in = company_x.shape[1]
    hid = W_cffn.shape[1]
    e_c = company_edge_index_h.shape[1]
    e_t = transaction_edge_index_h.shape[1]
    assert e_c == e_t and transaction_x.shape[0] == n
    seq_len = 50
    half = Whh_f.shape[1]

    kb = 4
    gtot = _cdiv(_cdiv(e_c, CH), NS * kb)
    g0 = (gtot * 28) // 40
    g1 = gtot - g0
    nch = NS * gtot * kb
    cpw = nch // NW
    e_pad = nch * CH

    bn = 2000
    grid = n // bn
    assert n % bn == 0
    n_pad = _cdiv(n + 8, NS * 8) * NS * 8

    pad = e_pad - e_c
    scrap = n + (jnp.arange(pad, dtype=I32) % (n_pad - n))

    def prep(ei):
        rows = jnp.concatenate([ei[0], scrap]).reshape(nch, CH)
        colsg = jnp.concatenate(
            [ei[1], jnp.zeros((pad,), I32)]).reshape(nch, CH)
        return rows, colsg

    rows_c, colsg_c = prep(company_edge_index_h)
    rows_t, colsg_t = prep(transaction_edge_index_h)

    nch_real = e_c // CH
    assert nch_real * CH == e_c
    degc_p, degt_p = _make_deg_kernel(n_pad, cpw, nch_real)(
        colsg_c, colsg_t)
    degc_p = jnp.transpose(degc_p.reshape(NC, n_pad))
    degt_p = jnp.transpose(degt_p.reshape(NC, n_pad))

    nh = (n, hid)
    tc1 = pl.pallas_call(
        _tc1_body,
        grid=(grid,),
        in_specs=[
            _row_spec(bn, (n, d_in)), _full_spec(W_cffn.shape),
            _full_spec(b_cffn.shape), _full_spec(W_cconv.shape),
            _full_spec(b_cconv.shape),
            _row_spec(bn, (n, d_in)), _full_spec(W_tffn.shape),
            _full_spec(b_tffn.shape), _full_spec(W_tc1.shape),
            _full_spec(b_tc1.shape),
            _row_spec(bn, (n_pad, 2)), _row_spec(bn, (n_pad, 2)),
        ],
        out_specs=[_row_spec(bn, nh), _row_spec(bn, nh),
                   _row_spec(bn, nh), _row_spec(bn, nh),
                   _row_spec(bn, (n, 1)), _row_spec(bn, (n, 1))],
        out_shape=[
            jax.ShapeDtypeStruct(nh, F32),
            jax.ShapeDtypeStruct(nh, F32),
            jax.ShapeDtypeStruct(nh, F32),
            jax.ShapeDtypeStruct(nh, F32),
            jax.ShapeDtypeStruct((n, 1), F32),
            jax.ShapeDtypeStruct((n, 1), F32),
        ],
    )
    x_c, y_c, x_t, y_1, dis_c, dis_t = tc1(
        company_x, W_cffn, b_cffn, W_cconv, b_cconv,
        transaction_x, W_tffn, b_tffn, W_tc1, b_tc1, degc_p, degt_p)

    sc_p, s1_p = _make_scat_kernel(2, n_pad, hid, g0, g1)(
        rows_c, colsg_c, y_c, rows_t, colsg_t, y_1)
    sc_p = sc_p.reshape(NC, n_pad, hid)[:, :n]
    s1_p = s1_p.reshape(NC, n_pad, hid)[:, :n]

    pspec = _part_spec(bn, hid)
    tc2 = pl.pallas_call(
        _tc2_body,
        grid=(grid,),
        in_specs=[
            _row_spec(bn, nh), _row_spec(bn, nh), pspec,
            _row_spec(bn, (n, 1)), _full_spec(W_cpost.shape),
            _full_spec(b_cpost.shape),
            _row_spec(bn, nh), _row_spec(bn, nh), pspec,
            _row_spec(bn, (n, 1)), _full_spec(W_tc2.shape),
            _full_spec(b_tc2.shape),
        ],
        out_specs=[_row_spec(bn, nh)] * 4,
        out_shape=[jax.ShapeDtypeStruct(nh, F32)] * 4,
    )
    x_cf, x_1, x_t2, y_2 = tc2(
        x_c, y_c, sc_p, dis_c, W_cpost, b_cpost,
        x_t, y_1, s1_p, dis_t, W_tc2, b_tc2)

    (s2_p,) = _make_scat_kernel(1, n_pad, hid, g0, g1)(
        rows_t, colsg_t, y_2)
    s2_p = s2_p.reshape(NC, n_pad, hid)[:, :n]

    wih_f_t = jnp.transpose(Wih_f)
    whh_f_t = jnp.transpose(Whh_f)
    wih_r_t = jnp.transpose(Wih_r)
    whh_r_t = jnp.transpose(Whh_r)
    b_f = bih_f + bhh_f
    b_r = bih_r + bhh_r
    tc3 = pl.pallas_call(
        functools.partial(_tc3_body, seq_len, half),
        grid=(grid,),
        in_specs=[
            pspec, _row_spec(bn, nh),
            _row_spec(bn, (n, 1)), _row_spec(bn, nh), _row_spec(bn, nh),
            _full_spec(W_tpost.shape), _full_spec(b_tpost.shape),
            _row_spec(bn, nh),
            _full_spec(wih_f_t.shape), _full_spec(whh_f_t.shape),
            _full_spec(b_f.shape),
            _full_spec(wih_r_t.shape), _full_spec(whh_r_t.shape),
            _full_spec(b_r.shape),
            _full_spec(W_attn.shape), _full_spec(b_attn.shape),
            _full_spec(W_fcc.shape), _full_spec(b_fcc.shape),
            _full_spec(W_fct.shape), _full_spec(b_fct.shape),
        ],
        out_specs=[_row_spec(bn, (n, 2)), _row_spec(bn, (n, 2)),
                   _row_spec(bn, nh)],
        out_shape=[
            jax.ShapeDtypeStruct((n, 2), F32),
            jax.ShapeDtypeStruct((n, 2), F32),
            jax.ShapeDtypeStruct(nh, F32),
        ],
        scratch_shapes=[pltpu.VMEM((seq_len + 6, hid), F32),
                        pltpu.VMEM((1, hid), F32)],
    )
    out_c, out_t, _ = tc3(
        s2_p, y_2, dis_t, x_t2, x_1, W_tpost, b_tpost, x_cf,
        wih_f_t, whh_f_t, b_f, wih_r_t, whh_r_t, b_r,
        W_attn, b_attn, W_fcc, b_fcc, W_fct, b_fct)
    return (out_c, out_t)
